# Initial kernel scaffold; baseline (speedup 1.0000x reference)
import jax
import jax.numpy as jnp
from jax.experimental import pallas as pl


def kernel(x, user_emb, movie_emb, W1, b1, W2, b2):
    raise NotImplementedError("write your pallas kernel here")


# speedup vs baseline: 2.5227x; 2.5227x over previous
import functools

import jax
import jax.numpy as jnp
from jax import lax
from jax.experimental import pallas as pl
from jax.experimental.pallas import tpu as pltpu
from jax.experimental.pallas import tpu_sc as plsc

_B = 16384
_D = 128
_H = 1024
_NC = 2
_NS = 16
_NW = _NC * _NS
_BPW = _B // _NW
_CH = 128
_NCH = _BPW // _CH

_BLK = 2048


def _gather_body(uidx_hbm, vidx_hbm, user_hbm, movie_hbm, u_out, v_out,
                 idx_v, rows_v, sem):
    wid = lax.axis_index("s") * _NC + lax.axis_index("c")
    base = wid * _BPW
    row0 = wid * _NCH
    for tbl, idx_hbm, out in ((user_hbm, uidx_hbm, u_out),
                              (movie_hbm, vidx_hbm, v_out)):
        pltpu.sync_copy(idx_hbm.at[pl.ds(row0, _NCH)], idx_v)
        copies = [
            pltpu.async_copy(tbl.at[idx_v.at[c]],
                             rows_v.at[pl.ds(c * _CH, _CH)], sem)
            for c in range(_NCH)
        ]
        for cp in copies:
            cp.wait()
        pltpu.sync_copy(rows_v, out.at[pl.ds(base, _BPW)])


@jax.jit
def _gather(uidx, vidx, user_emb, movie_emb):
    mesh = plsc.VectorSubcoreMesh(core_axis_name="c", subcore_axis_name="s",
                                  num_cores=_NC, num_subcores=_NS)
    return pl.kernel(
        _gather_body,
        out_type=[jax.ShapeDtypeStruct((_B, _D), jnp.float32),
                  jax.ShapeDtypeStruct((_B, _D), jnp.float32)],
        mesh=mesh,
        scratch_types=[
            pltpu.VMEM((_NCH, _CH), jnp.int32),
            pltpu.VMEM((_BPW, _D), jnp.float32),
            pltpu.SemaphoreType.DMA,
        ],
    )(uidx, vidx, user_emb, movie_emb)


def _mlp_body(u_ref, v_ref, w1a_ref, w1b_ref, b1_ref, w2_ref, b2_ref,
              out_ref):
    u = jnp.maximum(u_ref[...], 0.0).astype(jnp.bfloat16)
    v = jnp.maximum(v_ref[...], 0.0).astype(jnp.bfloat16)
    h = jnp.dot(u, w1a_ref[...], preferred_element_type=jnp.float32)
    h = h + jnp.dot(v, w1b_ref[...], preferred_element_type=jnp.float32)
    h = jnp.maximum(h + b1_ref[...], 0.0)
    logit = jnp.sum(h * w2_ref[...], axis=1, keepdims=True) + b2_ref[0, 0]
    out_ref[...] = 5.5 * jax.nn.sigmoid(-logit)


@jax.jit
def _mlp(U, V, w1a, w1b, b1, w2, b2):
    grid = (_B // _BLK,)
    return pl.pallas_call(
        _mlp_body,
        grid=grid,
        in_specs=[
            pl.BlockSpec((_BLK, _D), lambda i: (i, 0)),
            pl.BlockSpec((_BLK, _D), lambda i: (i, 0)),
            pl.BlockSpec((_D, _H), lambda i: (0, 0)),
            pl.BlockSpec((_D, _H), lambda i: (0, 0)),
            pl.BlockSpec((1, _H), lambda i: (0, 0)),
            pl.BlockSpec((1, _H), lambda i: (0, 0)),
            pl.BlockSpec((1, 1), lambda i: (0, 0)),
        ],
        out_specs=pl.BlockSpec((_BLK, 1), lambda i: (i, 0)),
        out_shape=jax.ShapeDtypeStruct((_B, 1), jnp.float32),
    )(U, V, w1a, w1b, b1, w2, b2)


def kernel(x, user_emb, movie_emb, W1, b1, W2, b2):
    uidx = x[:, 0].reshape(_B // _CH, _CH)
    vidx = x[:, 1].reshape(_B // _CH, _CH)
    U, V = _gather(uidx, vidx, user_emb, movie_emb)
    w1a = W1[:_D].astype(jnp.bfloat16)
    w1b = W1[_D:].astype(jnp.bfloat16)
    out = _mlp(U, V, w1a, w1b, b1.reshape(1, _H), W2.reshape(1, _H),
               b2.reshape(1, 1))
    return out.reshape(-1)


# speedup vs baseline: 2.9226x; 1.1585x over previous
import functools

import jax
import jax.numpy as jnp
from jax import lax
from jax.experimental import pallas as pl
from jax.experimental.pallas import tpu as pltpu
from jax.experimental.pallas import tpu_sc as plsc

_B = 16384
_D = 128
_H = 1024
_NC = 2
_NS = 16
_NW = _NC * _NS
_BPW = _B // _NW
_CH = 128
_NCH = _BPW // _CH

_BLK = 2048


def _gather_body(uidx_hbm, vidx_hbm, user_hbm, movie_hbm, h_out,
                 idx_v, rows_v, sem):
    wid = lax.axis_index("s") * _NC + lax.axis_index("c")
    base = wid * _BPW
    row0 = wid * _NCH
    for col, idx_hbm, tbl in ((0, uidx_hbm, user_hbm),
                              (_D, vidx_hbm, movie_hbm)):
        pltpu.sync_copy(idx_hbm.at[pl.ds(row0, _NCH)], idx_v)
        copies = [
            pltpu.async_copy(tbl.at[idx_v.at[c]],
                             rows_v.at[pl.ds(c * _CH, _CH)], sem)
            for c in range(_NCH)
        ]
        for cp in copies:
            cp.wait()
        pltpu.sync_copy(rows_v, h_out.at[pl.ds(base, _BPW), pl.ds(col, _D)])


@jax.jit
def _gather(uidx, vidx, user_emb, movie_emb):
    mesh = plsc.VectorSubcoreMesh(core_axis_name="c", subcore_axis_name="s",
                                  num_cores=_NC, num_subcores=_NS)
    return pl.kernel(
        _gather_body,
        out_type=jax.ShapeDtypeStruct((_B, 2 * _D), jnp.float32),
        mesh=mesh,
        scratch_types=[
            pltpu.VMEM((_NCH, _CH), jnp.int32),
            pltpu.VMEM((_BPW, _D), jnp.float32),
            pltpu.SemaphoreType.DMA,
        ],
    )(uidx, vidx, user_emb, movie_emb)


def _mlp_body(h_ref, w1_ref, b1_ref, w2_ref, b2_ref, out_ref):
    hin = jnp.maximum(h_ref[...].astype(jnp.bfloat16), 0)
    h = jnp.dot(hin, w1_ref[...], preferred_element_type=jnp.float32)
    h = jnp.maximum(h + b1_ref[...], 0.0)
    logit = jnp.sum(h * w2_ref[...], axis=1, keepdims=True) + b2_ref[0, 0]
    out_ref[...] = 5.5 * jax.nn.sigmoid(-logit)


@jax.jit
def _mlp(H, w1, b1, w2, b2):
    grid = (_B // _BLK,)
    return pl.pallas_call(
        _mlp_body,
        grid=grid,
        in_specs=[
            pl.BlockSpec((_BLK, 2 * _D), lambda i: (i, 0)),
            pl.BlockSpec((2 * _D, _H), lambda i: (0, 0)),
            pl.BlockSpec((1, _H), lambda i: (0, 0)),
            pl.BlockSpec((1, _H), lambda i: (0, 0)),
            pl.BlockSpec((1, 1), lambda i: (0, 0)),
        ],
        out_specs=pl.BlockSpec((_BLK, 1), lambda i: (i, 0)),
        out_shape=jax.ShapeDtypeStruct((_B, 1), jnp.float32),
    )(H, w1, b1, w2, b2)


def kernel(x, user_emb, movie_emb, W1, b1, W2, b2):
    uidx = x[:, 0].reshape(_B // _CH, _CH)
    vidx = x[:, 1].reshape(_B // _CH, _CH)
    H = _gather(uidx, vidx, user_emb, movie_emb)
    out = _mlp(H, W1.astype(jnp.bfloat16), b1.reshape(1, _H),
               W2.reshape(1, _H), b2.reshape(1, 1))
    return out.reshape(-1)


# speedup vs baseline: 3.0130x; 1.0309x over previous
import functools

import jax
import jax.numpy as jnp
from jax import lax
from jax.experimental import pallas as pl
from jax.experimental.pallas import tpu as pltpu
from jax.experimental.pallas import tpu_sc as plsc

_B = 16384
_D = 128
_H = 1024
_NC = 2
_NS = 16
_NW = _NC * _NS
_BPW = _B // _NW
_CH = 128
_NCH = _BPW // _CH

_BLK = 2048


def _gather_body(uidx_hbm, vidx_hbm, user_hbm, movie_hbm, h_out,
                 idx_v, rows_v, sem):
    wid = lax.axis_index("s") * _NC + lax.axis_index("c")
    base = wid * _BPW
    row0 = wid * _NCH
    for col, idx_hbm, tbl in ((0, uidx_hbm, user_hbm),
                              (_D, vidx_hbm, movie_hbm)):
        pltpu.sync_copy(idx_hbm.at[pl.ds(row0, _NCH)], idx_v)
        copies = [
            pltpu.async_copy(tbl.at[idx_v.at[c]],
                             rows_v.at[pl.ds(c * _CH, _CH)], sem)
            for c in range(_NCH)
        ]
        for cp in copies:
            cp.wait()
        pltpu.sync_copy(rows_v, h_out.at[pl.ds(base, _BPW), pl.ds(col, _D)])


@jax.jit
def _gather(uidx, vidx, user_emb, movie_emb):
    mesh = plsc.VectorSubcoreMesh(core_axis_name="c", subcore_axis_name="s",
                                  num_cores=_NC, num_subcores=_NS)
    return pl.kernel(
        _gather_body,
        out_type=jax.ShapeDtypeStruct((_B, 2 * _D), jnp.float32),
        mesh=mesh,
        scratch_types=[
            pltpu.VMEM((_NCH, _CH), jnp.int32),
            pltpu.VMEM((_BPW, _D), jnp.float32),
            pltpu.SemaphoreType.DMA,
        ],
    )(uidx, vidx, user_emb, movie_emb)


def _mlp_body(h_ref, w1_ref, b1_ref, w2_ref, b2_ref, out_ref):
    hin = jnp.maximum(h_ref[...].astype(jnp.bfloat16), 0)
    h = jnp.dot(hin, w1_ref[...], preferred_element_type=jnp.float32)
    h = jnp.maximum(h + b1_ref[...], 0.0).astype(jnp.bfloat16)
    logit = jax.lax.dot_general(w2_ref[...], h,
                                (((1,), (1,)), ((), ())),
                                preferred_element_type=jnp.float32)
    logit = logit + b2_ref[0, 0]
    out_ref[...] = 5.5 * jax.nn.sigmoid(-logit)


@jax.jit
def _mlp(H, w1, b1, w2, b2):
    grid = (_B // _BLK,)
    return pl.pallas_call(
        _mlp_body,
        grid=grid,
        in_specs=[
            pl.BlockSpec((_BLK, 2 * _D), lambda i: (i, 0)),
            pl.BlockSpec((2 * _D, _H), lambda i: (0, 0)),
            pl.BlockSpec((1, _H), lambda i: (0, 0)),
            pl.BlockSpec((1, _H), lambda i: (0, 0)),
            pl.BlockSpec((1, 1), lambda i: (0, 0)),
        ],
        out_specs=pl.BlockSpec((1, _BLK), lambda i: (0, i)),
        out_shape=jax.ShapeDtypeStruct((1, _B), jnp.float32),
    )(H, w1, b1, w2, b2)


def kernel(x, user_emb, movie_emb, W1, b1, W2, b2):
    uidx = x[:, 0].reshape(_B // _CH, _CH)
    vidx = x[:, 1].reshape(_B // _CH, _CH)
    H = _gather(uidx, vidx, user_emb, movie_emb)
    out = _mlp(H, W1.astype(jnp.bfloat16), b1.reshape(1, _H),
               W2.reshape(1, _H).astype(jnp.bfloat16), b2.reshape(1, 1))
    return out.reshape(_B)
